# Initial kernel scaffold; baseline (speedup 1.0000x reference)
import jax
import jax.numpy as jnp
from jax.experimental import pallas as pl


def kernel(x, edge_index, edge_weight):
    raise NotImplementedError("write your pallas kernel here")


# speedup vs baseline: 14.8104x; 14.8104x over previous
import functools

import jax
import jax.numpy as jnp
from jax import lax
from jax.experimental import pallas as pl
from jax.experimental.pallas import tpu as pltpu
from jax.experimental.pallas import tpu_sc as plsc

N = 10000
E = 320000
D = 128
NPAD = 10240
NC = 2
NS = 16
NW = NC * NS
CHUNK = 128
N_CHUNKS = E // CHUNK
TRIPS = (N_CHUNKS + NW - 1) // NW

_mesh = plsc.VectorSubcoreMesh(
    core_axis_name="c", subcore_axis_name="s", num_cores=NC, num_subcores=NS
)


def _deg_body(dst_hbm, ew_hbm, deg_out, deg_spmem, idx_v, ew_v, zbuf):
  cid = lax.axis_index("c")
  sid = lax.axis_index("s")
  wid = sid * NC + cid

  def zb(i, _):
    zbuf[pl.ds(i * 16, 16)] = jnp.zeros((16,), jnp.float32)
    return 0

  lax.fori_loop(0, 640 // 16, zb, 0)
  pltpu.sync_copy(zbuf, deg_spmem.at[pl.ds(sid * 640, 640)])
  plsc.subcore_barrier()

  def chunk_body(t, _):
    ch = wid + NW * t

    @pl.when(ch < N_CHUNKS)
    def _():
      base = ch * CHUNK
      pltpu.sync_copy(dst_hbm.at[pl.ds(base, CHUNK)], idx_v)
      pltpu.sync_copy(ew_hbm.at[pl.ds(base, CHUNK)], ew_v)
      pltpu.sync_copy(ew_v, deg_spmem.at[idx_v], add=True)

    return 0

  lax.fori_loop(0, TRIPS, chunk_body, 0)
  plsc.subcore_barrier()
  pltpu.sync_copy(
      deg_spmem.at[pl.ds(sid * 640, 640)],
      deg_out.at[pl.ds(cid * NPAD + sid * 640, 640)],
  )


_deg_call = pl.kernel(
    _deg_body,
    out_type=jax.ShapeDtypeStruct((NC * NPAD,), jnp.float32),
    mesh=_mesh,
    scratch_types=[
        pltpu.VMEM_SHARED((NPAD,), jnp.float32),
        pltpu.VMEM((CHUNK,), jnp.int32),
        pltpu.VMEM((CHUNK,), jnp.float32),
        pltpu.VMEM((640,), jnp.float32),
    ],
)


def _dinv_body(degp_ref, dinv_ref):
  deg = degp_ref[0] + degp_ref[1]
  good = deg > 0.0
  safe = jnp.where(good, deg, 1.0)
  dinv_ref[...] = jnp.where(good, lax.rsqrt(safe), 0.0)


_dinv_call = pl.pallas_call(
    _dinv_body,
    out_shape=jax.ShapeDtypeStruct((NPAD // D, D), jnp.float32),
)


NROWS = 10240
ROWS_PER_TILE = NROWS // NS
ZROWS = 128


def _prop_body(x_hbm, src_hbm, dst_hbm, ew_hbm, dinv_hbm, acc_out,
               acc_spmem, dinv_c, buf, src_v, dst_v, ew_v, zbuf, sem, sem2):
  cid = lax.axis_index("c")
  sid = lax.axis_index("s")
  wid = sid * NC + cid

  def zrow(r, _):
    for k in range(D // 16):
      zbuf[r, pl.ds(k * 16, 16)] = jnp.zeros((16,), jnp.float32)
    return 0

  lax.fori_loop(0, ZROWS, zrow, 0)
  for k in range(ROWS_PER_TILE // ZROWS):
    pltpu.sync_copy(
        zbuf, acc_spmem.at[pl.ds(sid * ROWS_PER_TILE + k * ZROWS, ZROWS)]
    )
  plsc.subcore_barrier()

  def chunk_body(t, _):
    ch = wid + NW * t

    @pl.when(ch < N_CHUNKS)
    def _():
      base = ch * CHUNK
      pltpu.sync_copy(src_hbm.at[pl.ds(base, CHUNK)], src_v)
      pltpu.sync_copy(dst_hbm.at[pl.ds(base, CHUNK)], dst_v)
      pltpu.sync_copy(ew_hbm.at[pl.ds(base, CHUNK)], ew_v)
      cp1 = pltpu.async_copy(x_hbm.at[src_v], buf, sem)
      cp2 = pltpu.async_copy(dinv_hbm.at[src_v], dinv_c, sem2)
      cp1.wait()
      cp2.wait()

      def g_body(g, _):
        dv = dinv_c[pl.ds(g * 16, 16)]
        ev = ew_v[pl.ds(g * 16, 16)]
        cvec = ev * dv
        for lane in range(16):
          s = cvec[lane]
          r = g * 16 + lane
          for k in range(D // 16):
            buf[r, pl.ds(k * 16, 16)] = buf[r, pl.ds(k * 16, 16)] * s
        return 0

      lax.fori_loop(0, CHUNK // 16, g_body, 0)
      pltpu.sync_copy(buf, acc_spmem.at[dst_v], add=True)

    return 0

  lax.fori_loop(0, TRIPS, chunk_body, 0)
  plsc.subcore_barrier()
  pltpu.sync_copy(
      acc_spmem.at[pl.ds(sid * ROWS_PER_TILE, ROWS_PER_TILE)],
      acc_out.at[cid, pl.ds(sid * ROWS_PER_TILE, ROWS_PER_TILE)],
  )


_prop_call = pl.kernel(
    _prop_body,
    out_type=jax.ShapeDtypeStruct((NC, NROWS, D), jnp.float32),
    mesh=_mesh,
    scratch_types=[
        pltpu.VMEM_SHARED((NROWS, D), jnp.float32),
        pltpu.VMEM((CHUNK,), jnp.float32),
        pltpu.VMEM((CHUNK, D), jnp.float32),
        pltpu.VMEM((CHUNK,), jnp.int32),
        pltpu.VMEM((CHUNK,), jnp.int32),
        pltpu.VMEM((CHUNK,), jnp.float32),
        pltpu.VMEM((ZROWS, D), jnp.float32),
        pltpu.SemaphoreType.DMA,
        pltpu.SemaphoreType.DMA,
    ],
)


def _combine_body(acc_ref, dinv_ref, out_ref):
  s = (acc_ref[0] + acc_ref[1]) * dinv_ref[...]
  out_ref[...] = s[:N]


_combine_call = pl.pallas_call(
    _combine_body,
    out_shape=jax.ShapeDtypeStruct((N, D), jnp.float32),
)


@jax.jit
def kernel(x, edge_index, edge_weight):
  src = edge_index[0].astype(jnp.int32)
  dst = edge_index[1].astype(jnp.int32)
  ew = edge_weight.astype(jnp.float32)
  deg_p = _deg_call(dst, ew)
  dinv2d = _dinv_call(deg_p.reshape(NC, NPAD // D, D))
  dinv_flat = dinv2d.reshape(NPAD)
  acc = _prop_call(x, src, dst, ew, dinv_flat)
  dinv_col = dinv_flat[:NROWS].reshape(NROWS, 1)
  return _combine_call(acc, dinv_col)
